# baseline (speedup 1.0000x reference)
import functools

import jax
import jax.numpy as jnp
from jax import lax
from jax.experimental import pallas as pl
from jax.experimental.pallas import tpu as pltpu
from jax.experimental.pallas import tpu_sc as plsc

_L = 16


@functools.partial(jax.jit, static_argnums=(4, 5))
def _edge_vals(table, row_i, col_i, ninv, chunk, nw):
    e_total = row_i.shape[0]
    n_nodes, d_feat = table.shape
    per_w = e_total // nw
    n_chunks = per_w // chunk
    mesh = plsc.VectorSubcoreMesh(core_axis_name="c", subcore_axis_name="s")

    @functools.partial(
        pl.kernel,
        out_type=jax.ShapeDtypeStruct((e_total,), jnp.float32),
        mesh=mesh,
        scratch_types=[
            pltpu.VMEM((chunk,), jnp.int32),
            pltpu.VMEM((chunk,), jnp.int32),
            pltpu.VMEM((chunk, d_feat), jnp.float32),
            pltpu.VMEM((chunk, d_feat), jnp.float32),
            pltpu.VMEM((chunk,), jnp.float32),
            pltpu.VMEM((_L,), jnp.float32),
            pltpu.SemaphoreType.DMA,
            pltpu.SemaphoreType.DMA,
        ],
        compiler_params=pltpu.CompilerParams(needs_layout_passes=False),
    )
    def k(table_h, row_h, col_h, ninv_h, out_h,
          idx_r, idx_c, rows_r, rows_c, vbuf, ninv_v, sem, sem2):
        wid = lax.axis_index("s") * mesh.num_cores + lax.axis_index("c")
        pltpu.sync_copy(ninv_h, ninv_v)
        ninv_vec = ninv_v[...]
        base_w = wid * per_w

        def chunk_body(ci, _):
            base = base_w + ci * chunk
            pltpu.sync_copy(row_h.at[pl.ds(base, chunk)], idx_r)
            pltpu.sync_copy(col_h.at[pl.ds(base, chunk)], idx_c)
            cp_r = pltpu.async_copy(table_h.at[idx_r], rows_r, sem)
            cp_c = pltpu.async_copy(table_h.at[idx_c], rows_c, sem2)
            cp_r.wait()
            cp_c.wait()

            lane_last = lax.iota(jnp.int32, _L) == (_L - 1)

            def edge_body(e, _):
                acc = jnp.zeros((_L,), jnp.float32)
                for dk in range(d_feat // _L):
                    a = rows_r[e, pl.ds(dk * _L, _L)]
                    b = rows_c[e, pl.ds(dk * _L, _L)]
                    dd = a - b
                    acc = acc + dd * dd
                tot = plsc.cumsum(acc)
                plsc.store_scatter(vbuf, [jnp.full((_L,), e, jnp.int32)],
                                   tot, mask=lane_last)
                return 0

            lax.fori_loop(0, chunk, edge_body, 0)

            def exp_body(g, _):
                v = vbuf[pl.ds(g * _L, _L)]
                vbuf[pl.ds(g * _L, _L)] = jnp.exp(v * ninv_vec)
                return 0

            lax.fori_loop(0, chunk // _L, exp_body, 0)
            pltpu.sync_copy(vbuf, out_h.at[pl.ds(base, chunk)])
            return 0

        lax.fori_loop(0, n_chunks, chunk_body, 0)

    return k(table, row_i, col_i, ninv)


def kernel(inputs, row, col, sigma):
    e_total = row.shape[0]
    nw = 32
    chunk = 400
    block = nw * chunk
    row_i = row.astype(jnp.int32)
    col_i = col.astype(jnp.int32)
    e_pad = ((e_total + block - 1) // block) * block
    if e_pad != e_total:
        row_i = jnp.pad(row_i, (0, e_pad - e_total))
        col_i = jnp.pad(col_i, (0, e_pad - e_total))
    ninv = jnp.full((_L,), -1.0 / (sigma * sigma), jnp.float32)
    vals = _edge_vals(inputs, row_i, col_i, ninv, chunk, nw)
    if e_pad != e_total:
        vals = vals[:e_total]
    return (row, col, vals)


# speedup vs baseline: 1.2988x; 1.2988x over previous
import functools

import jax
import jax.numpy as jnp
from jax import lax
from jax.experimental import pallas as pl
from jax.experimental.pallas import tpu as pltpu
from jax.experimental.pallas import tpu_sc as plsc

_L = 16


@functools.partial(jax.jit, static_argnums=(4, 5))
def _edge_vals(table, row_i, col_i, ninv, chunk, nw):
    e_total = row_i.shape[0]
    n_nodes, d_feat = table.shape
    per_w = e_total // nw
    n_chunks = per_w // chunk
    mesh = plsc.VectorSubcoreMesh(core_axis_name="c", subcore_axis_name="s")

    @functools.partial(
        pl.kernel,
        out_type=jax.ShapeDtypeStruct((e_total,), jnp.float32),
        mesh=mesh,
        scratch_types=[
            pltpu.VMEM((chunk,), jnp.int32),
            pltpu.VMEM((chunk,), jnp.int32),
            pltpu.VMEM((chunk, d_feat), jnp.float32),
            pltpu.VMEM((chunk, d_feat), jnp.float32),
            pltpu.VMEM((chunk,), jnp.float32),
            pltpu.VMEM((_L,), jnp.float32),
            pltpu.SemaphoreType.DMA,
            pltpu.SemaphoreType.DMA,
        ],
        compiler_params=pltpu.CompilerParams(needs_layout_passes=False),
    )
    def k(table_h, row_h, col_h, ninv_h, out_h,
          idx_r, idx_c, rows_r, rows_c, vbuf, ninv_v, sem, sem2):
        wid = lax.axis_index("s") * mesh.num_cores + lax.axis_index("c")
        pltpu.sync_copy(ninv_h, ninv_v)
        ninv_vec = ninv_v[...]
        base_w = wid * per_w

        def chunk_body(ci, _):
            base = base_w + ci * chunk
            pltpu.sync_copy(row_h.at[pl.ds(base, chunk)], idx_r)
            pltpu.sync_copy(col_h.at[pl.ds(base, chunk)], idx_c)
            cp_r = pltpu.async_copy(table_h.at[idx_r], rows_r, sem)
            cp_c = pltpu.async_copy(table_h.at[idx_c], rows_c, sem2)
            cp_r.wait()
            cp_c.wait()

            lane_last = lax.iota(jnp.int32, _L) == (_L - 1)

            @plsc.parallel_loop(0, chunk, unroll=4)
            def edge_body(e):
                acc = jnp.zeros((_L,), jnp.float32)
                for dk in range(d_feat // _L):
                    a = rows_r[e, pl.ds(dk * _L, _L)]
                    b = rows_c[e, pl.ds(dk * _L, _L)]
                    dd = a - b
                    acc = acc + dd * dd
                tot = plsc.cumsum(acc)
                plsc.store_scatter(vbuf, [jnp.full((_L,), e, jnp.int32)],
                                   tot, mask=lane_last)

            @plsc.parallel_loop(0, chunk // _L, unroll=4)
            def exp_body(g):
                v = vbuf[pl.ds(g * _L, _L)]
                vbuf[pl.ds(g * _L, _L)] = jnp.exp(v * ninv_vec)
            pltpu.sync_copy(vbuf, out_h.at[pl.ds(base, chunk)])
            return 0

        lax.fori_loop(0, n_chunks, chunk_body, 0)

    return k(table, row_i, col_i, ninv)


def kernel(inputs, row, col, sigma):
    e_total = row.shape[0]
    nw = 32
    chunk = 400
    block = nw * chunk
    row_i = row.astype(jnp.int32)
    col_i = col.astype(jnp.int32)
    e_pad = ((e_total + block - 1) // block) * block
    if e_pad != e_total:
        row_i = jnp.pad(row_i, (0, e_pad - e_total))
        col_i = jnp.pad(col_i, (0, e_pad - e_total))
    ninv = jnp.full((_L,), -1.0 / (sigma * sigma), jnp.float32)
    vals = _edge_vals(inputs, row_i, col_i, ninv, chunk, nw)
    if e_pad != e_total:
        vals = vals[:e_total]
    return (row, col, vals)


# speedup vs baseline: 1.5355x; 1.1822x over previous
import functools

import jax
import jax.numpy as jnp
from jax import lax
from jax.experimental import pallas as pl
from jax.experimental.pallas import tpu as pltpu
from jax.experimental.pallas import tpu_sc as plsc

_L = 16


@functools.partial(jax.jit, static_argnums=(4, 5))
def _edge_vals(table, row_i, col_i, ninv, chunk, nw):
    e_total = row_i.shape[0]
    n_nodes, d_feat = table.shape
    per_w = e_total // nw
    n_chunks = per_w // chunk
    mesh = plsc.VectorSubcoreMesh(core_axis_name="c", subcore_axis_name="s")

    @functools.partial(
        pl.kernel,
        out_type=jax.ShapeDtypeStruct((e_total,), jnp.float32),
        mesh=mesh,
        scratch_types=[
            pltpu.VMEM((chunk,), jnp.int32),
            pltpu.VMEM((chunk,), jnp.int32),
            pltpu.VMEM((chunk,), jnp.int32),
            pltpu.VMEM((chunk,), jnp.int32),
            pltpu.VMEM((chunk, d_feat), jnp.float32),
            pltpu.VMEM((chunk, d_feat), jnp.float32),
            pltpu.VMEM((chunk, d_feat), jnp.float32),
            pltpu.VMEM((chunk, d_feat), jnp.float32),
            pltpu.VMEM((((chunk + _L - 1) // _L) * _L,), jnp.float32),
            pltpu.VMEM((_L,), jnp.float32),
            pltpu.SemaphoreType.DMA,
            pltpu.SemaphoreType.DMA,
            pltpu.SemaphoreType.DMA,
            pltpu.SemaphoreType.DMA,
        ],
        compiler_params=pltpu.CompilerParams(needs_layout_passes=False),
    )
    def k(table_h, row_h, col_h, ninv_h, out_h,
          idx_ra, idx_ca, idx_rb, idx_cb,
          rows_ra, rows_ca, rows_rb, rows_cb,
          vbuf, ninv_v, sem_a, sem_a2, sem_b, sem_b2):
        wid = lax.axis_index("s") * mesh.num_cores + lax.axis_index("c")
        pltpu.sync_copy(ninv_h, ninv_v)
        ninv_vec = ninv_v[...]
        base_w = wid * per_w
        lane_last = lax.iota(jnp.int32, _L) == (_L - 1)

        def stage(ci, idx_r, idx_c, rows_r, rows_c, sem, sem2):
            base = base_w + ci * chunk
            pltpu.sync_copy(row_h.at[pl.ds(base, chunk)], idx_r)
            pltpu.sync_copy(col_h.at[pl.ds(base, chunk)], idx_c)
            pltpu.async_copy(table_h.at[idx_r], rows_r, sem)
            pltpu.async_copy(table_h.at[idx_c], rows_c, sem2)

        def wait_gathers(idx_r, idx_c, rows_r, rows_c, sem, sem2):
            pltpu.make_async_copy(table_h.at[idx_r], rows_r, sem).wait()
            pltpu.make_async_copy(table_h.at[idx_c], rows_c, sem2).wait()

        def compute(ci, rows_r, rows_c):
            @plsc.parallel_loop(0, chunk, unroll=4)
            def edge_body(e):
                acc = jnp.zeros((_L,), jnp.float32)
                for dk in range(d_feat // _L):
                    a = rows_r[e, pl.ds(dk * _L, _L)]
                    b = rows_c[e, pl.ds(dk * _L, _L)]
                    dd = a - b
                    acc = acc + dd * dd
                tot = plsc.cumsum(acc)
                plsc.store_scatter(vbuf, [jnp.full((_L,), e, jnp.int32)],
                                   tot, mask=lane_last)

            @plsc.parallel_loop(0, (chunk + _L - 1) // _L, unroll=4)
            def exp_body(g):
                v = vbuf[pl.ds(g * _L, _L)]
                vbuf[pl.ds(g * _L, _L)] = jnp.exp(v * ninv_vec)
            pltpu.sync_copy(vbuf.at[pl.ds(0, chunk)],
                            out_h.at[pl.ds(base_w + ci * chunk, chunk)])

        set_a = (idx_ra, idx_ca, rows_ra, rows_ca, sem_a, sem_a2)
        set_b = (idx_rb, idx_cb, rows_rb, rows_cb, sem_b, sem_b2)

        stage(0, *set_a)

        def body2(i, _):
            c0 = 2 * i
            stage(c0 + 1, *set_b)
            wait_gathers(*set_a)
            compute(c0, rows_ra, rows_ca)
            stage(jnp.minimum(c0 + 2, n_chunks - 1), *set_a)
            wait_gathers(*set_b)
            compute(c0 + 1, rows_rb, rows_cb)
            return 0

        lax.fori_loop(0, n_chunks // 2, body2, 0)
        wait_gathers(*set_a)
        if n_chunks % 2 == 1:
            compute(n_chunks - 1, rows_ra, rows_ca)

    return k(table, row_i, col_i, ninv)


def kernel(inputs, row, col, sigma):
    e_total = row.shape[0]
    nw = 32
    chunk = 200
    block = nw * chunk
    row_i = row.astype(jnp.int32)
    col_i = col.astype(jnp.int32)
    e_pad = ((e_total + block - 1) // block) * block
    if e_pad != e_total:
        row_i = jnp.pad(row_i, (0, e_pad - e_total))
        col_i = jnp.pad(col_i, (0, e_pad - e_total))
    ninv = jnp.full((_L,), -1.0 / (sigma * sigma), jnp.float32)
    vals = _edge_vals(inputs, row_i, col_i, ninv, chunk, nw)
    if e_pad != e_total:
        vals = vals[:e_total]
    return (row, col, vals)


# speedup vs baseline: 1.5385x; 1.0019x over previous
import functools

import jax
import jax.numpy as jnp
from jax import lax
from jax.experimental import pallas as pl
from jax.experimental.pallas import tpu as pltpu
from jax.experimental.pallas import tpu_sc as plsc

_L = 16


@functools.partial(jax.jit, static_argnums=(4, 5))
def _edge_vals(table, row_i, col_i, ninv, chunk, nw):
    e_total = row_i.shape[0]
    n_nodes, d_feat = table.shape
    per_w = e_total // nw
    n_chunks = per_w // chunk
    mesh = plsc.VectorSubcoreMesh(core_axis_name="c", subcore_axis_name="s")

    @functools.partial(
        pl.kernel,
        out_type=jax.ShapeDtypeStruct((e_total,), jnp.float32),
        mesh=mesh,
        scratch_types=[
            pltpu.VMEM((chunk,), jnp.int32),
            pltpu.VMEM((chunk,), jnp.int32),
            pltpu.VMEM((chunk,), jnp.int32),
            pltpu.VMEM((chunk,), jnp.int32),
            pltpu.VMEM((chunk, d_feat), jnp.float32),
            pltpu.VMEM((chunk, d_feat), jnp.float32),
            pltpu.VMEM((chunk, d_feat), jnp.float32),
            pltpu.VMEM((chunk, d_feat), jnp.float32),
            pltpu.VMEM((((chunk + _L - 1) // _L) * _L,), jnp.float32),
            pltpu.VMEM((_L,), jnp.float32),
            pltpu.SemaphoreType.DMA,
            pltpu.SemaphoreType.DMA,
            pltpu.SemaphoreType.DMA,
            pltpu.SemaphoreType.DMA,
        ],
        compiler_params=pltpu.CompilerParams(needs_layout_passes=False),
    )
    def k(table_h, row_h, col_h, ninv_h, out_h,
          idx_ra, idx_ca, idx_rb, idx_cb,
          rows_ra, rows_ca, rows_rb, rows_cb,
          vbuf, ninv_v, sem_a, sem_a2, sem_b, sem_b2):
        wid = lax.axis_index("s") * mesh.num_cores + lax.axis_index("c")
        pltpu.sync_copy(ninv_h, ninv_v)
        ninv_vec = ninv_v[...]
        base_w = wid * per_w
        lane_last = lax.iota(jnp.int32, _L) == (_L - 1)

        def stage(ci, idx_r, idx_c, rows_r, rows_c, sem, sem2):
            base = base_w + ci * chunk
            pltpu.sync_copy(row_h.at[pl.ds(base, chunk)], idx_r)
            pltpu.sync_copy(col_h.at[pl.ds(base, chunk)], idx_c)
            pltpu.async_copy(table_h.at[idx_r], rows_r, sem)
            pltpu.async_copy(table_h.at[idx_c], rows_c, sem2)

        def wait_gathers(idx_r, idx_c, rows_r, rows_c, sem, sem2):
            pltpu.make_async_copy(table_h.at[idx_r], rows_r, sem).wait()
            pltpu.make_async_copy(table_h.at[idx_c], rows_c, sem2).wait()

        def compute(ci, rows_r, rows_c):
            @plsc.parallel_loop(0, chunk, unroll=8)
            def edge_body(e):
                acc = jnp.zeros((_L,), jnp.float32)
                for dk in range(d_feat // _L):
                    a = rows_r[e, pl.ds(dk * _L, _L)]
                    b = rows_c[e, pl.ds(dk * _L, _L)]
                    dd = a - b
                    acc = acc + dd * dd
                tot = plsc.cumsum(acc)
                plsc.store_scatter(vbuf, [jnp.full((_L,), e, jnp.int32)],
                                   tot, mask=lane_last)

            @plsc.parallel_loop(0, (chunk + _L - 1) // _L, unroll=4)
            def exp_body(g):
                v = vbuf[pl.ds(g * _L, _L)]
                vbuf[pl.ds(g * _L, _L)] = jnp.exp(v * ninv_vec)
            pltpu.sync_copy(vbuf.at[pl.ds(0, chunk)],
                            out_h.at[pl.ds(base_w + ci * chunk, chunk)])

        set_a = (idx_ra, idx_ca, rows_ra, rows_ca, sem_a, sem_a2)
        set_b = (idx_rb, idx_cb, rows_rb, rows_cb, sem_b, sem_b2)

        stage(0, *set_a)

        def body2(i, _):
            c0 = 2 * i
            stage(c0 + 1, *set_b)
            wait_gathers(*set_a)
            compute(c0, rows_ra, rows_ca)
            stage(jnp.minimum(c0 + 2, n_chunks - 1), *set_a)
            wait_gathers(*set_b)
            compute(c0 + 1, rows_rb, rows_cb)
            return 0

        lax.fori_loop(0, n_chunks // 2, body2, 0)
        wait_gathers(*set_a)
        if n_chunks % 2 == 1:
            compute(n_chunks - 1, rows_ra, rows_ca)

    return k(table, row_i, col_i, ninv)


def kernel(inputs, row, col, sigma):
    e_total = row.shape[0]
    nw = 32
    chunk = 200
    block = nw * chunk
    row_i = row.astype(jnp.int32)
    col_i = col.astype(jnp.int32)
    e_pad = ((e_total + block - 1) // block) * block
    if e_pad != e_total:
        row_i = jnp.pad(row_i, (0, e_pad - e_total))
        col_i = jnp.pad(col_i, (0, e_pad - e_total))
    ninv = jnp.full((_L,), -1.0 / (sigma * sigma), jnp.float32)
    vals = _edge_vals(inputs, row_i, col_i, ninv, chunk, nw)
    if e_pad != e_total:
        vals = vals[:e_total]
    return (row, col, vals)
